# Initial kernel scaffold; baseline (speedup 1.0000x reference)
import jax
import jax.numpy as jnp
from jax.experimental import pallas as pl


def kernel(x):
    raise NotImplementedError("write your pallas kernel here")


# speedup vs baseline: 3.0352x; 3.0352x over previous
import jax
import jax.numpy as jnp
from jax.experimental import pallas as pl


def _decode_kernel(x_ref, o_ref, *, img_w, img_h):
    cx = x_ref[0, 0, 0, 0]
    cy = x_ref[0, 1, 0, 0]
    dw = x_ref[0, 2, 0, 0] * 0.5
    dh = x_ref[0, 3, 0, 0] * 0.5
    score = x_ref[0, 4, 0, 0]
    x1 = jnp.clip(cx - dw, 0.0, img_w)
    y1 = jnp.clip(cy - dh, 0.0, img_h)
    x2 = jnp.clip(cx + dw, 0.0, img_w)
    y2 = jnp.clip(cy + dh, 0.0, img_h)
    row = jax.lax.broadcasted_iota(jnp.int32, (8, 128), 0)
    col = jax.lax.broadcasted_iota(jnp.int32, (8, 128), 1)
    out = jnp.zeros((8, 128), jnp.float32)
    for i, v in enumerate((x1, y1, x2, y2, score)):
        out = jnp.where((row == 0) & (col == i + 1), v, out)
    o_ref[...] = out


def kernel(x):
    img_h, img_w = float(x.shape[2]), float(x.shape[3])
    import functools
    body = functools.partial(_decode_kernel, img_w=img_w, img_h=img_h)
    tile = pl.pallas_call(
        body,
        grid=(1,),
        in_specs=[pl.BlockSpec((1, 8, 8, 128), lambda i: (0, 0, 0, 0))],
        out_specs=pl.BlockSpec((8, 128), lambda i: (0, 0)),
        out_shape=jax.ShapeDtypeStruct((8, 128), jnp.float32),
    )(x)
    return tile[0:1, 0:7]


# speedup vs baseline: 3.1120x; 1.0253x over previous
import jax
import jax.numpy as jnp
from jax.experimental import pallas as pl


def _decode_kernel(x_ref, o_ref, *, img_w, img_h):
    cx = x_ref[0, 0, 0, 0]
    cy = x_ref[0, 1, 0, 0]
    dw = x_ref[0, 2, 0, 0] * 0.5
    dh = x_ref[0, 3, 0, 0] * 0.5
    score = x_ref[0, 4, 0, 0]
    x1 = jnp.clip(cx - dw, 0.0, img_w)
    y1 = jnp.clip(cy - dh, 0.0, img_h)
    x2 = jnp.clip(cx + dw, 0.0, img_w)
    y2 = jnp.clip(cy + dh, 0.0, img_h)
    col = jax.lax.broadcasted_iota(jnp.int32, (1, 7), 1)
    out = jnp.zeros((1, 7), jnp.float32)
    for i, v in enumerate((x1, y1, x2, y2, score)):
        out = jnp.where(col == i + 1, v, out)
    o_ref[...] = out


def kernel(x):
    img_h, img_w = float(x.shape[2]), float(x.shape[3])
    import functools
    body = functools.partial(_decode_kernel, img_w=img_w, img_h=img_h)
    return pl.pallas_call(
        body,
        grid=(1,),
        in_specs=[pl.BlockSpec((1, 8, 8, 128), lambda i: (0, 0, 0, 0))],
        out_specs=pl.BlockSpec((1, 7), lambda i: (0, 0)),
        out_shape=jax.ShapeDtypeStruct((1, 7), jnp.float32),
    )(x)
